# baseline (speedup 1.0000x reference)
import functools
import jax
import jax.numpy as jnp
from jax import lax
from jax.experimental import pallas as pl
from jax.experimental.pallas import tpu as pltpu
from jax.experimental.pallas import tpu_sc as plsc

_N = 10000
_E = 320000
_HID = 128
_HEADS = 8
_HD = 16
_NC = 2
_NS = 16
_NW = _NC * _NS
_EPW = _E // _NW
_B = 40
_BP = 48
_NB = _EPW // _B
_NPAD = 10240
_RPS = _NPAD // _NS
_SROW = _NPAD // 8


def _proj_body(x_ref, win_ref, wq_ref, wk_ref, wv_ref, h_ref, q_ref, kv_ref):
    h = jnp.dot(x_ref[...], win_ref[...], preferred_element_type=jnp.float32)
    h_ref[...] = h
    q_ref[...] = jnp.dot(h, wq_ref[...], preferred_element_type=jnp.float32)
    k = jnp.dot(h, wk_ref[...], preferred_element_type=jnp.float32)
    v = jnp.dot(h, wv_ref[...], preferred_element_type=jnp.float32)
    kv_ref[...] = jnp.concatenate([k, v], axis=1)


def _edge_proj_body(a_ref, we_ref, e_ref):
    e_ref[...] = jnp.dot(a_ref[...], we_ref[...],
                         preferred_element_type=jnp.float32)


_GDN = lax.GatherDimensionNumbers(offset_dims=(), collapsed_slice_dims=(0,),
                                  start_index_map=(0,))


def _shuffle(x, idx):
    return lax.gather(x, idx[:, None], _GDN, slice_sizes=(1,),
                      mode=lax.GatherScatterMode.PROMISE_IN_BOUNDS)

def _sc_edge_body(q_hbm, kv_hbm, e_hbm, src_hbm, dst_hbm,
                  out_hbm, out2_hbm,
                  sd_v, dst8_v, q_v, kv_v, em_v, sc_v,
                  acc_sh, sacc_sh, sem, sem2):
    c = lax.axis_index("c")
    s = lax.axis_index("s")
    ebase = (c * _NS + s) * _EPW

    zero16 = jnp.zeros((16,), jnp.float32)
    lane = lax.iota(jnp.int32, 16)

    def _zsc(i, _):
        for j in range(_HID // 16):
            sc_v[i, pl.ds(j * 16, 16)] = zero16
        return 0
    lax.fori_loop(0, _BP, _zsc, 0)

    def _zrow(i, _):
        for j in range(_HID // 16):
            em_v[i, pl.ds(j * 16, 16)] = zero16
        return 0
    lax.fori_loop(0, _BP, _zrow, 0)
    for r in range(_RPS // _B):
        pltpu.sync_copy(em_v.at[pl.ds(0, _B), :],
                        acc_sh.at[pl.ds(s * _RPS + r * _B, _B), :])
    for r in range(_SROW // _NS // _B):
        pltpu.sync_copy(em_v.at[pl.ds(0, _B), :],
                        sacc_sh.at[pl.ds(s * (_SROW // _NS) + r * _B, _B), :])
    plsc.subcore_barrier()

    def _batch(b, _):
        base = ebase + b * _B
        sd_v[0, pl.ds(32, 16)] = lane * 0
        sd_v[1, pl.ds(32, 16)] = lane * 0
        pltpu.sync_copy(src_hbm.at[pl.ds(base, _B)],
                        sd_v.at[0, pl.ds(0, _B)])
        pltpu.sync_copy(dst_hbm.at[pl.ds(base, _B)],
                        sd_v.at[1, pl.ds(0, _B)])
        cp_q = pltpu.async_copy(q_hbm.at[sd_v.at[1]], q_v, sem)
        cp_kv = pltpu.async_copy(kv_hbm.at[sd_v.at[0]], kv_v, sem)
        cp_e = pltpu.async_copy(e_hbm.at[pl.ds(base, _B), :],
                                em_v.at[pl.ds(0, _B), :], sem)
        for g in range(_BP // 16):
            dst8_v[pl.ds(g * 16, 16)] = lax.shift_right_logical(
                sd_v[1, pl.ds(g * 16, 16)], 3)
        cp_q.wait()
        cp_kv.wait()
        cp_e.wait()

        @plsc.parallel_loop(0, _B, 1, unroll=4)
        def _edge(i):
            g = (i // 16) * 16
            j = i - g
            dchunk = sd_v[1, pl.ds(g, 16)]
            d_bc = _shuffle(dchunk, lane * 0 + j)
            scores = zero16
            for h in range(_HEADS):
                qd = q_v[i, pl.ds(h * _HD, _HD)]
                eh = em_v[i, pl.ds(h * _HD, _HD)]
                ke = kv_v[i, pl.ds(h * _HD, _HD)] + eh
                prod = qd * ke
                for st in (1, 2, 4, 8):
                    prod = prod + _shuffle(prod, lane ^ st)
                sc = jnp.maximum(prod, 0.0) * 0.25
                ve = kv_v[i, pl.ds(_HID + h * _HD, _HD)] + eh
                em_v[i, pl.ds(h * _HD, _HD)] = sc * ve
                scores = jnp.where(lane == h, sc, scores)
            for j2 in range(_HID // 16):
                sc_v[i, pl.ds(j2 * 16, 16)] = zero16
            plsc.store_scatter(sc_v, [lane * 0 + i, (d_bc & 7) * 16 + lane],
                               scores)

        cpm = pltpu.async_copy(em_v, acc_sh.at[sd_v.at[1]], sem2, add=True)
        cps = pltpu.async_copy(sc_v, sacc_sh.at[dst8_v], sem2, add=True)
        cpm.wait()
        cps.wait()
        return 0
    lax.fori_loop(0, _NB, _batch, 0)

    plsc.subcore_barrier()
    pltpu.sync_copy(acc_sh.at[pl.ds(s * _RPS, _RPS), :],
                    out_hbm.at[c, pl.ds(s * _RPS, _RPS), :])
    pltpu.sync_copy(sacc_sh.at[pl.ds(s * (_SROW // _NS), _SROW // _NS), :],
                    out2_hbm.at[c, pl.ds(s * (_SROW // _NS), _SROW // _NS), :])


def _final_body(p_ref, p2_ref, h_ref, wout_ref, bout_ref, gnw_ref, gnb_ref,
                gna_ref, wcls_ref, bcls_ref, out_ref):
    unnorm = p_ref[0, :_N, :] + p_ref[1, :_N, :]
    rs = p2_ref[0, :_N, :] + p2_ref[1, :_N, :]
    col = lax.broadcasted_iota(jnp.int32, (16, _HID), 1)
    row = lax.broadcasted_iota(jnp.int32, (16, _HID), 0)
    sel = jnp.where((col // _HD) == row, 1.0, 0.0).astype(jnp.float32)
    denom = jnp.dot(rs, sel, preferred_element_type=jnp.float32) + 1e-6
    agg = unnorm / denom
    out = jnp.dot(agg, wout_ref[...], preferred_element_type=jnp.float32)
    out = out + bout_ref[...] + h_ref[...]
    mean = jnp.mean(out, axis=0, keepdims=True)
    shifted = out - gna_ref[...] * mean
    var = jnp.mean(shifted * shifted, axis=0, keepdims=True)
    out = gnw_ref[...] * shifted / jnp.sqrt(var + 1e-5) + gnb_ref[...]
    out = jnp.maximum(out, 0.0)
    out_ref[...] = jnp.dot(out, wcls_ref[...],
                           preferred_element_type=jnp.float32) + bcls_ref[...]


def kernel(x, edge_index, edge_attr, W_in, WQ, WK, WV, WE, W_out, b_out,
           gn_weight, gn_bias, gn_alpha, W_cls, b_cls):
    f32 = jnp.float32

    h, q, kv = pl.pallas_call(
        _proj_body,
        out_shape=[
            jax.ShapeDtypeStruct((_N, _HID), f32),
            jax.ShapeDtypeStruct((_N, _HID), f32),
            jax.ShapeDtypeStruct((_N, 2 * _HID), f32),
        ],
    )(x, W_in, WQ, WK, WV)

    e = pl.pallas_call(
        _edge_proj_body,
        grid=(16,),
        in_specs=[
            pl.BlockSpec((_E // 16, 16), lambda i: (i, 0)),
            pl.BlockSpec((16, _HID), lambda i: (0, 0)),
        ],
        out_specs=pl.BlockSpec((_E // 16, _HID), lambda i: (i, 0)),
        out_shape=jax.ShapeDtypeStruct((_E, _HID), f32),
    )(edge_attr, WE)

    sc_edge = functools.partial(
        pl.kernel,
        mesh=plsc.VectorSubcoreMesh(core_axis_name="c", subcore_axis_name="s"),
        out_type=[
            jax.ShapeDtypeStruct((_NC, _NPAD, _HID), f32),
            jax.ShapeDtypeStruct((_NC, _SROW, _HID), f32),
        ],
        scratch_types=[
            pltpu.VMEM((2, _BP), jnp.int32),
            pltpu.VMEM((_BP,), jnp.int32),
            pltpu.VMEM((_BP, _HID), f32),
            pltpu.VMEM((_BP, 2 * _HID), f32),
            pltpu.VMEM((_BP, _HID), f32),
            pltpu.VMEM((_BP, _HID), f32),
            pltpu.VMEM_SHARED((_NPAD, _HID), f32),
            pltpu.VMEM_SHARED((_SROW, _HID), f32),
            pltpu.SemaphoreType.DMA,
            pltpu.SemaphoreType.DMA,
        ],
        compiler_params=pltpu.CompilerParams(needs_layout_passes=False),
    )(_sc_edge_body)
    partials, partials2 = sc_edge(q, kv, e, edge_index[0], edge_index[1])

    logits_pad = pl.pallas_call(
        _final_body,
        out_shape=jax.ShapeDtypeStruct((_N, _HID), f32),
    )(partials, partials2.reshape(_NC, _NPAD, 16), h, W_out,
      b_out.reshape(1, _HID),
      gn_weight.reshape(1, _HID), gn_bias.reshape(1, _HID),
      gn_alpha.reshape(1, _HID),
      jnp.pad(W_cls, ((0, 0), (0, _HID - 2))),
      jnp.pad(b_cls, (0, _HID - 2)).reshape(1, _HID))

    return logits_pad[:, :2]


# speedup vs baseline: 1.5160x; 1.5160x over previous
import functools
import jax
import jax.numpy as jnp
from jax import lax
from jax.experimental import pallas as pl
from jax.experimental.pallas import tpu as pltpu
from jax.experimental.pallas import tpu_sc as plsc

_N = 10000
_E = 320000
_HID = 128
_HEADS = 8
_HD = 16
_NC = 2
_NS = 16
_NW = _NC * _NS
_EPW = _E // _NW
_B = 16
_NB = _EPW // _B
_NPAD = 10240
_RPS = _NPAD // _NS
_SROW = _NPAD // 8


def _proj_body(x_ref, win_ref, wq_ref, wk_ref, wv_ref, h_ref, q_ref, kv_ref):
    h = jnp.dot(x_ref[...], win_ref[...], preferred_element_type=jnp.float32)
    h_ref[...] = h
    q_ref[...] = jnp.dot(h, wq_ref[...], preferred_element_type=jnp.float32)
    k = jnp.dot(h, wk_ref[...], preferred_element_type=jnp.float32)
    v = jnp.dot(h, wv_ref[...], preferred_element_type=jnp.float32)
    kv_ref[...] = jnp.concatenate([k, v], axis=1)


def _edge_proj_body(a_ref, we_ref, e_ref):
    e_ref[...] = jnp.dot(a_ref[...], we_ref[...],
                         preferred_element_type=jnp.float32)


_GDN = lax.GatherDimensionNumbers(offset_dims=(), collapsed_slice_dims=(0,),
                                  start_index_map=(0,))


def _shuffle(x, idx):
    return lax.gather(x, idx[:, None], _GDN, slice_sizes=(1,),
                      mode=lax.GatherScatterMode.PROMISE_IN_BOUNDS)

def _sc_edge_body(q_hbm, kv_hbm, e_hbm, src_hbm, dst_hbm,
                  out_hbm, out2_hbm,
                  src_a, src_b, dst_a, dst_b, d8_a, d8_b,
                  q_a, q_b, kv_a, kv_b, em_a, em_b, sc_a, sc_b,
                  acc_sh, sacc_sh, gs_a, gs_b, ss_a, ss_b):
    c = lax.axis_index("c")
    s = lax.axis_index("s")
    ebase = (c * _NS + s) * _EPW

    zero16 = jnp.zeros((16,), jnp.float32)
    lane = lax.iota(jnp.int32, 16)

    slot_a = (src_a, dst_a, d8_a, q_a, kv_a, em_a, sc_a, gs_a, ss_a)
    slot_b = (src_b, dst_b, d8_b, q_b, kv_b, em_b, sc_b, gs_b, ss_b)

    def _zb(i, _):
        for buf in (em_a, em_b, sc_a, sc_b):
            for j in range(_HID // 16):
                buf[i, pl.ds(j * 16, 16)] = zero16
        return 0
    lax.fori_loop(0, _B, _zb, 0)

    tgts = [acc_sh.at[pl.ds(s * _RPS + r * _B, _B), :]
            for r in range(_RPS // _B)]
    tgts += [sacc_sh.at[pl.ds(s * (_SROW // _NS) + r * _B, _B), :]
             for r in range(_SROW // _NS // _B)]
    for i0 in range(0, len(tgts), 8):
        cps = [pltpu.async_copy(em_a, t, gs_a) for t in tgts[i0:i0 + 8]]
        for cp in cps:
            cp.wait()
    plsc.subcore_barrier()

    def _prefetch(b, slot):
        src_v, dst_v, d8_v, q_v, kv_v, em_v, sc_v, gsem, ssem = slot
        base = ebase + b * _B
        pltpu.sync_copy(src_hbm.at[pl.ds(base, _B)], src_v)
        pltpu.sync_copy(dst_hbm.at[pl.ds(base, _B)], dst_v)
        d8_v[...] = lax.shift_right_logical(dst_v[...], 3)
        pltpu.async_copy(q_hbm.at[dst_v], q_v, gsem)
        pltpu.async_copy(kv_hbm.at[src_v], kv_v, gsem)
        pltpu.async_copy(e_hbm.at[pl.ds(base, _B), :], em_v, gsem)

    def _wait_gathers(b, slot):
        src_v, dst_v, d8_v, q_v, kv_v, em_v, sc_v, gsem, ssem = slot
        base = ebase + b * _B
        pltpu.make_async_copy(q_hbm.at[dst_v], q_v, gsem).wait()
        pltpu.make_async_copy(kv_hbm.at[src_v], kv_v, gsem).wait()
        pltpu.make_async_copy(e_hbm.at[pl.ds(base, _B), :], em_v, gsem).wait()

    def _wait_scatters(slot):
        src_v, dst_v, d8_v, q_v, kv_v, em_v, sc_v, gsem, ssem = slot
        pltpu.make_async_copy(em_v, acc_sh.at[dst_v], ssem).wait()
        pltpu.make_async_copy(sc_v, sacc_sh.at[d8_v], ssem).wait()

    def _compute(slot):
        src_v, dst_v, d8_v, q_v, kv_v, em_v, sc_v, gsem, ssem = slot
        dchunk = dst_v[...]

        @plsc.parallel_loop(0, _B, 1, unroll=4)
        def _edge(i):
            d_bc = _shuffle(dchunk, lane * 0 + i)
            scores = zero16
            for h in range(_HEADS):
                qd = q_v[i, pl.ds(h * _HD, _HD)]
                eh = em_v[i, pl.ds(h * _HD, _HD)]
                ke = kv_v[i, pl.ds(h * _HD, _HD)] + eh
                prod = qd * ke
                for st in (1, 2, 4, 8):
                    prod = prod + _shuffle(prod, lane ^ st)
                sc = jnp.maximum(prod, 0.0) * 0.25
                ve = kv_v[i, pl.ds(_HID + h * _HD, _HD)] + eh
                em_v[i, pl.ds(h * _HD, _HD)] = sc * ve
                scores = jnp.where(lane == h, sc, scores)
            for j2 in range(_HID // 16):
                sc_v[i, pl.ds(j2 * 16, 16)] = zero16
            plsc.store_scatter(sc_v, [lane * 0 + i, (d_bc & 7) * 16 + lane],
                               scores)

        pltpu.async_copy(em_v, acc_sh.at[dst_v], ssem, add=True)
        pltpu.async_copy(sc_v, sacc_sh.at[d8_v], ssem, add=True)

    def _step(b, cur, nxt, guard):
        _wait_gathers(b, cur)
        _compute(cur)
        if guard is None:
            _wait_scatters(nxt)
        else:
            @pl.when(guard)
            def _():
                _wait_scatters(nxt)
        _prefetch(b + 1, nxt)

    _prefetch(0, slot_a)

    def _pair(k, _):
        b0 = k * 2
        _step(b0, slot_a, slot_b, k >= 1)
        _step(b0 + 1, slot_b, slot_a, None)
        return 0
    lax.fori_loop(0, (_NB - 1) // 2, _pair, 0)

    _wait_gathers(_NB - 1, slot_a)
    _compute(slot_a)
    _wait_scatters(slot_b)
    _wait_scatters(slot_a)

    plsc.subcore_barrier()
    pltpu.sync_copy(acc_sh.at[pl.ds(s * _RPS, _RPS), :],
                    out_hbm.at[c, pl.ds(s * _RPS, _RPS), :])
    pltpu.sync_copy(sacc_sh.at[pl.ds(s * (_SROW // _NS), _SROW // _NS), :],
                    out2_hbm.at[c, pl.ds(s * (_SROW // _NS), _SROW // _NS), :])


def _final_body(p_ref, p2_ref, h_ref, wout_ref, bout_ref, gnw_ref, gnb_ref,
                gna_ref, wcls_ref, bcls_ref, out_ref):
    unnorm = p_ref[0, :_N, :] + p_ref[1, :_N, :]
    rs = p2_ref[0, :_N, :] + p2_ref[1, :_N, :]
    col = lax.broadcasted_iota(jnp.int32, (16, _HID), 1)
    row = lax.broadcasted_iota(jnp.int32, (16, _HID), 0)
    sel = jnp.where((col // _HD) == row, 1.0, 0.0).astype(jnp.float32)
    denom = jnp.dot(rs, sel, preferred_element_type=jnp.float32) + 1e-6
    agg = unnorm / denom
    out = jnp.dot(agg, wout_ref[...], preferred_element_type=jnp.float32)
    out = out + bout_ref[...] + h_ref[...]
    mean = jnp.mean(out, axis=0, keepdims=True)
    shifted = out - gna_ref[...] * mean
    var = jnp.mean(shifted * shifted, axis=0, keepdims=True)
    out = gnw_ref[...] * shifted / jnp.sqrt(var + 1e-5) + gnb_ref[...]
    out = jnp.maximum(out, 0.0)
    out_ref[...] = jnp.dot(out, wcls_ref[...],
                           preferred_element_type=jnp.float32) + bcls_ref[...]


def kernel(x, edge_index, edge_attr, W_in, WQ, WK, WV, WE, W_out, b_out,
           gn_weight, gn_bias, gn_alpha, W_cls, b_cls):
    f32 = jnp.float32

    h, q, kv = pl.pallas_call(
        _proj_body,
        out_shape=[
            jax.ShapeDtypeStruct((_N, _HID), f32),
            jax.ShapeDtypeStruct((_N, _HID), f32),
            jax.ShapeDtypeStruct((_N, 2 * _HID), f32),
        ],
    )(x, W_in, WQ, WK, WV)

    e = pl.pallas_call(
        _edge_proj_body,
        grid=(16,),
        in_specs=[
            pl.BlockSpec((_E // 16, 16), lambda i: (i, 0)),
            pl.BlockSpec((16, _HID), lambda i: (0, 0)),
        ],
        out_specs=pl.BlockSpec((_E // 16, _HID), lambda i: (i, 0)),
        out_shape=jax.ShapeDtypeStruct((_E, _HID), f32),
    )(edge_attr, WE)

    sc_edge = functools.partial(
        pl.kernel,
        mesh=plsc.VectorSubcoreMesh(core_axis_name="c", subcore_axis_name="s"),
        out_type=[
            jax.ShapeDtypeStruct((_NC, _NPAD, _HID), f32),
            jax.ShapeDtypeStruct((_NC, _SROW, _HID), f32),
        ],
        scratch_types=(
            [pltpu.VMEM((_B,), jnp.int32)] * 6
            + [pltpu.VMEM((_B, _HID), f32)] * 2
            + [pltpu.VMEM((_B, 2 * _HID), f32)] * 2
            + [pltpu.VMEM((_B, _HID), f32)] * 4
            + [
                pltpu.VMEM_SHARED((_NPAD, _HID), f32),
                pltpu.VMEM_SHARED((_SROW, _HID), f32),
            ]
            + [pltpu.SemaphoreType.DMA] * 4
        ),
        compiler_params=pltpu.CompilerParams(needs_layout_passes=False),
    )(_sc_edge_body)
    partials, partials2 = sc_edge(q, kv, e, edge_index[0], edge_index[1])

    logits_pad = pl.pallas_call(
        _final_body,
        out_shape=jax.ShapeDtypeStruct((_N, _HID), f32),
    )(partials, partials2.reshape(_NC, _NPAD, 16), h, W_out,
      b_out.reshape(1, _HID),
      gn_weight.reshape(1, _HID), gn_bias.reshape(1, _HID),
      gn_alpha.reshape(1, _HID),
      jnp.pad(W_cls, ((0, 0), (0, _HID - 2))),
      jnp.pad(b_cls, (0, _HID - 2)).reshape(1, _HID))

    return logits_pad[:, :2]


# speedup vs baseline: 2.5044x; 1.6519x over previous
import functools
import jax
import jax.numpy as jnp
from jax import lax
from jax.experimental import pallas as pl
from jax.experimental.pallas import tpu as pltpu
from jax.experimental.pallas import tpu_sc as plsc

_N = 10000
_E = 320000
_HID = 128
_HEADS = 8
_HD = 16
_NC = 2
_NS = 16
_NW = _NC * _NS
_EPW = _E // _NW
_B = 48
_NB = 208
_BT = _EPW - _NB * _B
_NPAD = 10240
_RPS = _NPAD // _NS
_SROW = _NPAD // 8


def _proj_body(x_ref, win_ref, wq_ref, wk_ref, wv_ref, h_ref, q_ref, kv_ref):
    h = jnp.dot(x_ref[...], win_ref[...], preferred_element_type=jnp.float32)
    h_ref[...] = h
    q_ref[...] = jnp.dot(h, wq_ref[...], preferred_element_type=jnp.float32)
    k = jnp.dot(h, wk_ref[...], preferred_element_type=jnp.float32)
    v = jnp.dot(h, wv_ref[...], preferred_element_type=jnp.float32)
    kv_ref[...] = jnp.concatenate([k, v], axis=1)


def _edge_proj_body(a_ref, we_ref, e_ref):
    e_ref[...] = jnp.dot(a_ref[...], we_ref[...],
                         preferred_element_type=jnp.float32)


_GDN = lax.GatherDimensionNumbers(offset_dims=(), collapsed_slice_dims=(0,),
                                  start_index_map=(0,))


def _shuffle(x, idx):
    return lax.gather(x, idx[:, None], _GDN, slice_sizes=(1,),
                      mode=lax.GatherScatterMode.PROMISE_IN_BOUNDS)

def _sc_edge_body(q_hbm, kv_hbm, e_hbm, src_hbm, dst_hbm,
                  out_hbm, out2_hbm,
                  src_a, src_b, dst_a, dst_b, d8_a, d8_b,
                  src_t, dst_t, d8_t,
                  q_v, kv_v, em_v, sc_v,
                  acc_sh, sacc_sh, isem, gsem, ssem):
    c = lax.axis_index("c")
    s = lax.axis_index("s")
    ebase = (c * _NS + s) * _EPW

    zero16 = jnp.zeros((16,), jnp.float32)
    lane = lax.iota(jnp.int32, 16)

    idx_a = (src_a, dst_a, d8_a)
    idx_b = (src_b, dst_b, d8_b)

    def _zb(i, _):
        for buf in (em_v, sc_v):
            for j in range(_HID // 16):
                buf[i, pl.ds(j * 16, 16)] = zero16
        return 0
    lax.fori_loop(0, _B, _zb, 0)

    tgts = [acc_sh.at[pl.ds(s * _RPS + r * 40, 40), :]
            for r in range(_RPS // 40)]
    tgts += [sacc_sh.at[pl.ds(s * (_SROW // _NS) + r * 40, 40), :]
             for r in range(_SROW // _NS // 40)]
    for i0 in range(0, len(tgts), 6):
        cps = [pltpu.async_copy(em_v.at[pl.ds(0, 40), :], t, gsem)
               for t in tgts[i0:i0 + 6]]
        for cp in cps:
            cp.wait()
    plsc.subcore_barrier()

    def _idx_issue(b, idx, n):
        src_v, dst_v, d8_v = idx
        base = ebase + b * _B
        pltpu.async_copy(src_hbm.at[pl.ds(base, n)], src_v, isem)
        pltpu.async_copy(dst_hbm.at[pl.ds(base, n)], dst_v, isem)

    def _idx_wait(b, idx, n):
        src_v, dst_v, d8_v = idx
        base = ebase + b * _B
        pltpu.make_async_copy(src_hbm.at[pl.ds(base, n)], src_v, isem).wait()
        pltpu.make_async_copy(dst_hbm.at[pl.ds(base, n)], dst_v, isem).wait()
        for g in range(n // 16):
            d8_v[pl.ds(g * 16, 16)] = lax.shift_right_logical(
                dst_v[pl.ds(g * 16, 16)], 3)

    def _gather_issue(b, idx, n):
        src_v, dst_v, d8_v = idx
        base = ebase + b * _B
        pltpu.async_copy(q_hbm.at[dst_v], q_v.at[pl.ds(0, n), :], gsem)
        pltpu.async_copy(kv_hbm.at[src_v], kv_v.at[pl.ds(0, n), :], gsem)
        pltpu.async_copy(e_hbm.at[pl.ds(base, n), :],
                         em_v.at[pl.ds(0, n), :], gsem)

    def _gather_wait(b, idx, n):
        src_v, dst_v, d8_v = idx
        base = ebase + b * _B
        pltpu.make_async_copy(q_hbm.at[dst_v], q_v.at[pl.ds(0, n), :],
                              gsem).wait()
        pltpu.make_async_copy(kv_hbm.at[src_v], kv_v.at[pl.ds(0, n), :],
                              gsem).wait()
        pltpu.make_async_copy(e_hbm.at[pl.ds(base, n), :],
                              em_v.at[pl.ds(0, n), :], gsem).wait()

    def _compute(idx, n):
        src_v, dst_v, d8_v = idx

        @plsc.parallel_loop(0, n, 1, unroll=4)
        def _edge(i):
            g = (i // 16) * 16
            j = i - g
            dchunk = dst_v[pl.ds(g, 16)]
            d_bc = _shuffle(dchunk, lane * 0 + j)
            scores = zero16
            for h in range(_HEADS):
                qd = q_v[i, pl.ds(h * _HD, _HD)]
                eh = em_v[i, pl.ds(h * _HD, _HD)]
                ke = kv_v[i, pl.ds(h * _HD, _HD)] + eh
                prod = qd * ke
                for st in (1, 2, 4, 8):
                    prod = prod + _shuffle(prod, lane ^ st)
                sc = jnp.maximum(prod, 0.0) * 0.25
                ve = kv_v[i, pl.ds(_HID + h * _HD, _HD)] + eh
                em_v[i, pl.ds(h * _HD, _HD)] = sc * ve
                scores = jnp.where(lane == h, sc, scores)
            for j2 in range(_HID // 16):
                sc_v[i, pl.ds(j2 * 16, 16)] = zero16
            plsc.store_scatter(sc_v, [lane * 0 + i, (d_bc & 7) * 16 + lane],
                               scores)

        pltpu.async_copy(em_v.at[pl.ds(0, n), :], acc_sh.at[dst_v],
                         ssem, add=True)
        pltpu.async_copy(sc_v.at[pl.ds(0, n), :], sacc_sh.at[d8_v],
                         ssem, add=True)
        pltpu.make_async_copy(em_v.at[pl.ds(0, n), :], acc_sh.at[dst_v],
                              ssem).wait()
        pltpu.make_async_copy(sc_v.at[pl.ds(0, n), :], sacc_sh.at[d8_v],
                              ssem).wait()

    def _body(b, cur, nxt, nxt_n):
        if nxt is not None:
            _idx_issue(b + 1, nxt, nxt_n)
        _gather_wait(b, cur, _B)
        _compute(cur, _B)
        if nxt is not None:
            _idx_wait(b + 1, nxt, nxt_n)
            _gather_issue(b + 1, nxt, nxt_n)

    _idx_issue(0, idx_a, _B)
    _idx_wait(0, idx_a, _B)
    _gather_issue(0, idx_a, _B)

    def _pair(k, _):
        b0 = k * 2
        _body(b0, idx_a, idx_b, _B)
        _body(b0 + 1, idx_b, idx_a, _B)
        return 0
    lax.fori_loop(0, _NB // 2 - 1, _pair, 0)
    _body(_NB - 2, idx_a, idx_b, _B)
    _body(_NB - 1, idx_b, (src_t, dst_t, d8_t), _BT)
    _gather_wait(_NB, (src_t, dst_t, d8_t), _BT)
    _compute((src_t, dst_t, d8_t), _BT)

    plsc.subcore_barrier()
    pltpu.sync_copy(acc_sh.at[pl.ds(s * _RPS, _RPS), :],
                    out_hbm.at[c, pl.ds(s * _RPS, _RPS), :])
    pltpu.sync_copy(sacc_sh.at[pl.ds(s * (_SROW // _NS), _SROW // _NS), :],
                    out2_hbm.at[c, pl.ds(s * (_SROW // _NS), _SROW // _NS), :])


def _final_body(p_ref, p2_ref, h_ref, wout_ref, bout_ref, gnw_ref, gnb_ref,
                gna_ref, wcls_ref, bcls_ref, out_ref):
    unnorm = p_ref[0, :_N, :] + p_ref[1, :_N, :]
    rs = p2_ref[0, :_N, :] + p2_ref[1, :_N, :]
    col = lax.broadcasted_iota(jnp.int32, (16, _HID), 1)
    row = lax.broadcasted_iota(jnp.int32, (16, _HID), 0)
    sel = jnp.where((col // _HD) == row, 1.0, 0.0).astype(jnp.float32)
    denom = jnp.dot(rs, sel, preferred_element_type=jnp.float32) + 1e-6
    agg = unnorm / denom
    out = jnp.dot(agg, wout_ref[...], preferred_element_type=jnp.float32)
    out = out + bout_ref[...] + h_ref[...]
    mean = jnp.mean(out, axis=0, keepdims=True)
    shifted = out - gna_ref[...] * mean
    var = jnp.mean(shifted * shifted, axis=0, keepdims=True)
    out = gnw_ref[...] * shifted / jnp.sqrt(var + 1e-5) + gnb_ref[...]
    out = jnp.maximum(out, 0.0)
    out_ref[...] = jnp.dot(out, wcls_ref[...],
                           preferred_element_type=jnp.float32) + bcls_ref[...]


def kernel(x, edge_index, edge_attr, W_in, WQ, WK, WV, WE, W_out, b_out,
           gn_weight, gn_bias, gn_alpha, W_cls, b_cls):
    f32 = jnp.float32

    h, q, kv = pl.pallas_call(
        _proj_body,
        out_shape=[
            jax.ShapeDtypeStruct((_N, _HID), f32),
            jax.ShapeDtypeStruct((_N, _HID), f32),
            jax.ShapeDtypeStruct((_N, 2 * _HID), f32),
        ],
    )(x, W_in, WQ, WK, WV)

    e = pl.pallas_call(
        _edge_proj_body,
        grid=(16,),
        in_specs=[
            pl.BlockSpec((_E // 16, 16), lambda i: (i, 0)),
            pl.BlockSpec((16, _HID), lambda i: (0, 0)),
        ],
        out_specs=pl.BlockSpec((_E // 16, _HID), lambda i: (i, 0)),
        out_shape=jax.ShapeDtypeStruct((_E, _HID), f32),
    )(edge_attr, WE)

    sc_edge = functools.partial(
        pl.kernel,
        mesh=plsc.VectorSubcoreMesh(core_axis_name="c", subcore_axis_name="s"),
        out_type=[
            jax.ShapeDtypeStruct((_NC, _NPAD, _HID), f32),
            jax.ShapeDtypeStruct((_NC, _SROW, _HID), f32),
        ],
        scratch_types=(
            [pltpu.VMEM((_B,), jnp.int32)] * 6
            + [pltpu.VMEM((_BT,), jnp.int32)] * 3
            + [
                pltpu.VMEM((_B, _HID), f32),
                pltpu.VMEM((_B, 2 * _HID), f32),
                pltpu.VMEM((_B, _HID), f32),
                pltpu.VMEM((_B, _HID), f32),
                pltpu.VMEM_SHARED((_NPAD, _HID), f32),
                pltpu.VMEM_SHARED((_SROW, _HID), f32),
            ]
            + [pltpu.SemaphoreType.DMA] * 3
        ),
        compiler_params=pltpu.CompilerParams(needs_layout_passes=False),
    )(_sc_edge_body)
    partials, partials2 = sc_edge(q, kv, e, edge_index[0], edge_index[1])

    logits_pad = pl.pallas_call(
        _final_body,
        out_shape=jax.ShapeDtypeStruct((_N, _HID), f32),
    )(partials, partials2.reshape(_NC, _NPAD, 16), h, W_out,
      b_out.reshape(1, _HID),
      gn_weight.reshape(1, _HID), gn_bias.reshape(1, _HID),
      gn_alpha.reshape(1, _HID),
      jnp.pad(W_cls, ((0, 0), (0, _HID - 2))),
      jnp.pad(b_cls, (0, _HID - 2)).reshape(1, _HID))

    return logits_pad[:, :2]


# speedup vs baseline: 2.8145x; 1.1238x over previous
import functools
import jax
import jax.numpy as jnp
from jax import lax
from jax.experimental import pallas as pl
from jax.experimental.pallas import tpu as pltpu
from jax.experimental.pallas import tpu_sc as plsc

_N = 10000
_E = 320000
_HID = 128
_HEADS = 8
_HD = 16
_NC = 2
_NS = 16
_NW = _NC * _NS
_EPW = _E // _NW
_B = 48
_NB = 208
_BT = _EPW - _NB * _B
_NPAD = 10240
_RPS = _NPAD // _NS
_SROW = _NPAD // 8


def _proj_body(x_ref, win_ref, wq_ref, wk_ref, wv_ref, h_ref, q_ref, kv_ref):
    h = jnp.dot(x_ref[...], win_ref[...], preferred_element_type=jnp.float32)
    h_ref[...] = h
    q_ref[...] = jnp.dot(h, wq_ref[...], preferred_element_type=jnp.float32)
    k = jnp.dot(h, wk_ref[...], preferred_element_type=jnp.float32)
    v = jnp.dot(h, wv_ref[...], preferred_element_type=jnp.float32)
    kv_ref[...] = jnp.concatenate([k, v], axis=1)


def _edge_proj_body(a_ref, we_ref, e_ref):
    e_ref[...] = jnp.dot(a_ref[...], we_ref[...],
                         preferred_element_type=jnp.float32)


_GDN = lax.GatherDimensionNumbers(offset_dims=(), collapsed_slice_dims=(0,),
                                  start_index_map=(0,))


def _shuffle(x, idx):
    return lax.gather(x, idx[:, None], _GDN, slice_sizes=(1,),
                      mode=lax.GatherScatterMode.PROMISE_IN_BOUNDS)

def _sc_edge_body(q_hbm, kv_hbm, e_hbm, src_hbm, dst_hbm,
                  out_hbm, out2_hbm,
                  src_a, src_b, dst_a, dst_b, d8_a, d8_b,
                  src_t, dst_t, d8_t,
                  q_v, kv_v, em_v, sc_v,
                  acc_sh, sacc_sh, isem, gsem, ssem):
    c = lax.axis_index("c")
    s = lax.axis_index("s")
    ebase = (c * _NS + s) * _EPW

    zero16 = jnp.zeros((16,), jnp.float32)
    lane = lax.iota(jnp.int32, 16)

    idx_a = (src_a, dst_a, d8_a)
    idx_b = (src_b, dst_b, d8_b)

    def _zb(i, _):
        for buf in (em_v, sc_v):
            for j in range(_HID // 16):
                buf[i, pl.ds(j * 16, 16)] = zero16
        return 0
    lax.fori_loop(0, _B, _zb, 0)

    tgts = [acc_sh.at[pl.ds(s * _RPS + r * 40, 40), :]
            for r in range(_RPS // 40)]
    tgts += [sacc_sh.at[pl.ds(s * (_SROW // _NS) + r * 40, 40), :]
             for r in range(_SROW // _NS // 40)]
    for i0 in range(0, len(tgts), 6):
        cps = [pltpu.async_copy(em_v.at[pl.ds(0, 40), :], t, gsem)
               for t in tgts[i0:i0 + 6]]
        for cp in cps:
            cp.wait()
    plsc.subcore_barrier()

    def _idx_issue(b, idx, n):
        src_v, dst_v, d8_v = idx
        base = ebase + b * _B
        pltpu.async_copy(src_hbm.at[pl.ds(base, n)], src_v, isem)
        pltpu.async_copy(dst_hbm.at[pl.ds(base, n)], dst_v, isem)

    def _idx_wait(b, idx, n):
        src_v, dst_v, d8_v = idx
        base = ebase + b * _B
        pltpu.make_async_copy(src_hbm.at[pl.ds(base, n)], src_v, isem).wait()
        pltpu.make_async_copy(dst_hbm.at[pl.ds(base, n)], dst_v, isem).wait()
        for g in range(n // 16):
            d8_v[pl.ds(g * 16, 16)] = lax.shift_right_logical(
                dst_v[pl.ds(g * 16, 16)], 3)

    def _gather_issue(b, idx, n):
        src_v, dst_v, d8_v = idx
        base = ebase + b * _B
        pltpu.async_copy(q_hbm.at[dst_v], q_v.at[pl.ds(0, n), :], gsem)
        pltpu.async_copy(kv_hbm.at[src_v], kv_v.at[pl.ds(0, n), :], gsem)
        pltpu.async_copy(e_hbm.at[pl.ds(base, n), :],
                         em_v.at[pl.ds(0, n), :], gsem)

    def _gather_wait(b, idx, n):
        src_v, dst_v, d8_v = idx
        base = ebase + b * _B
        pltpu.make_async_copy(q_hbm.at[dst_v], q_v.at[pl.ds(0, n), :],
                              gsem).wait()
        pltpu.make_async_copy(kv_hbm.at[src_v], kv_v.at[pl.ds(0, n), :],
                              gsem).wait()
        pltpu.make_async_copy(e_hbm.at[pl.ds(base, n), :],
                              em_v.at[pl.ds(0, n), :], gsem).wait()

    def _compute(idx, n):
        src_v, dst_v, d8_v = idx

        @plsc.parallel_loop(0, n, 1, unroll=8)
        def _edge(i):
            g = (i // 16) * 16
            j = i - g
            dchunk = dst_v[pl.ds(g, 16)]
            d_bc = _shuffle(dchunk, lane * 0 + j)
            scores = zero16
            for h in range(_HEADS):
                qd = q_v[i, pl.ds(h * _HD, _HD)]
                eh = em_v[i, pl.ds(h * _HD, _HD)]
                ke = kv_v[i, pl.ds(h * _HD, _HD)] + eh
                prod = qd * ke
                for st in (1, 2, 4, 8):
                    prod = prod + _shuffle(prod, lane ^ st)
                sc = jnp.maximum(prod, 0.0) * 0.25
                ve = kv_v[i, pl.ds(_HID + h * _HD, _HD)] + eh
                em_v[i, pl.ds(h * _HD, _HD)] = sc * ve
                scores = jnp.where(lane == h, sc, scores)
            for j2 in range(_HID // 16):
                sc_v[i, pl.ds(j2 * 16, 16)] = zero16
            plsc.store_scatter(sc_v, [lane * 0 + i, (d_bc & 7) * 16 + lane],
                               scores)

        pltpu.async_copy(em_v.at[pl.ds(0, n), :], acc_sh.at[dst_v],
                         ssem, add=True)
        pltpu.async_copy(sc_v.at[pl.ds(0, n), :], sacc_sh.at[d8_v],
                         ssem, add=True)
        pltpu.make_async_copy(em_v.at[pl.ds(0, n), :], acc_sh.at[dst_v],
                              ssem).wait()
        pltpu.make_async_copy(sc_v.at[pl.ds(0, n), :], sacc_sh.at[d8_v],
                              ssem).wait()

    def _body(b, cur, nxt, nxt_n):
        if nxt is not None:
            _idx_issue(b + 1, nxt, nxt_n)
        _gather_wait(b, cur, _B)
        _compute(cur, _B)
        if nxt is not None:
            _idx_wait(b + 1, nxt, nxt_n)
            _gather_issue(b + 1, nxt, nxt_n)

    _idx_issue(0, idx_a, _B)
    _idx_wait(0, idx_a, _B)
    _gather_issue(0, idx_a, _B)

    def _pair(k, _):
        b0 = k * 2
        _body(b0, idx_a, idx_b, _B)
        _body(b0 + 1, idx_b, idx_a, _B)
        return 0
    lax.fori_loop(0, _NB // 2 - 1, _pair, 0)
    _body(_NB - 2, idx_a, idx_b, _B)
    _body(_NB - 1, idx_b, (src_t, dst_t, d8_t), _BT)
    _gather_wait(_NB, (src_t, dst_t, d8_t), _BT)
    _compute((src_t, dst_t, d8_t), _BT)

    plsc.subcore_barrier()
    pltpu.sync_copy(acc_sh.at[pl.ds(s * _RPS, _RPS), :],
                    out_hbm.at[c, pl.ds(s * _RPS, _RPS), :])
    pltpu.sync_copy(sacc_sh.at[pl.ds(s * (_SROW // _NS), _SROW // _NS), :],
                    out2_hbm.at[c, pl.ds(s * (_SROW // _NS), _SROW // _NS), :])


def _final_body(p_ref, p2_ref, h_ref, wout_ref, bout_ref, gnw_ref, gnb_ref,
                gna_ref, wcls_ref, bcls_ref, out_ref):
    unnorm = p_ref[0, :_N, :] + p_ref[1, :_N, :]
    rs = p2_ref[0, :_N, :] + p2_ref[1, :_N, :]
    col = lax.broadcasted_iota(jnp.int32, (16, _HID), 1)
    row = lax.broadcasted_iota(jnp.int32, (16, _HID), 0)
    sel = jnp.where((col // _HD) == row, 1.0, 0.0).astype(jnp.float32)
    denom = jnp.dot(rs, sel, preferred_element_type=jnp.float32) + 1e-6
    agg = unnorm / denom
    out = jnp.dot(agg, wout_ref[...], preferred_element_type=jnp.float32)
    out = out + bout_ref[...] + h_ref[...]
    mean = jnp.mean(out, axis=0, keepdims=True)
    shifted = out - gna_ref[...] * mean
    var = jnp.mean(shifted * shifted, axis=0, keepdims=True)
    out = gnw_ref[...] * shifted / jnp.sqrt(var + 1e-5) + gnb_ref[...]
    out = jnp.maximum(out, 0.0)
    out_ref[...] = jnp.dot(out, wcls_ref[...],
                           preferred_element_type=jnp.float32) + bcls_ref[...]


def kernel(x, edge_index, edge_attr, W_in, WQ, WK, WV, WE, W_out, b_out,
           gn_weight, gn_bias, gn_alpha, W_cls, b_cls):
    f32 = jnp.float32

    h, q, kv = pl.pallas_call(
        _proj_body,
        out_shape=[
            jax.ShapeDtypeStruct((_N, _HID), f32),
            jax.ShapeDtypeStruct((_N, _HID), f32),
            jax.ShapeDtypeStruct((_N, 2 * _HID), f32),
        ],
    )(x, W_in, WQ, WK, WV)

    e = pl.pallas_call(
        _edge_proj_body,
        grid=(16,),
        in_specs=[
            pl.BlockSpec((_E // 16, 16), lambda i: (i, 0)),
            pl.BlockSpec((16, _HID), lambda i: (0, 0)),
        ],
        out_specs=pl.BlockSpec((_E // 16, _HID), lambda i: (i, 0)),
        out_shape=jax.ShapeDtypeStruct((_E, _HID), f32),
    )(edge_attr, WE)

    sc_edge = functools.partial(
        pl.kernel,
        mesh=plsc.VectorSubcoreMesh(core_axis_name="c", subcore_axis_name="s"),
        out_type=[
            jax.ShapeDtypeStruct((_NC, _NPAD, _HID), f32),
            jax.ShapeDtypeStruct((_NC, _SROW, _HID), f32),
        ],
        scratch_types=(
            [pltpu.VMEM((_B,), jnp.int32)] * 6
            + [pltpu.VMEM((_BT,), jnp.int32)] * 3
            + [
                pltpu.VMEM((_B, _HID), f32),
                pltpu.VMEM((_B, 2 * _HID), f32),
                pltpu.VMEM((_B, _HID), f32),
                pltpu.VMEM((_B, _HID), f32),
                pltpu.VMEM_SHARED((_NPAD, _HID), f32),
                pltpu.VMEM_SHARED((_SROW, _HID), f32),
            ]
            + [pltpu.SemaphoreType.DMA] * 3
        ),
        compiler_params=pltpu.CompilerParams(needs_layout_passes=False),
    )(_sc_edge_body)
    partials, partials2 = sc_edge(q, kv, e, edge_index[0], edge_index[1])

    logits_pad = pl.pallas_call(
        _final_body,
        out_shape=jax.ShapeDtypeStruct((_N, _HID), f32),
    )(partials, partials2.reshape(_NC, _NPAD, 16), h, W_out,
      b_out.reshape(1, _HID),
      gn_weight.reshape(1, _HID), gn_bias.reshape(1, _HID),
      gn_alpha.reshape(1, _HID),
      jnp.pad(W_cls, ((0, 0), (0, _HID - 2))),
      jnp.pad(b_cls, (0, _HID - 2)).reshape(1, _HID))

    return logits_pad[:, :2]
